# baseline (speedup 1.0000x reference)
import functools

import jax
import jax.numpy as jnp
from jax import lax
from jax.experimental import pallas as pl
from jax.experimental.pallas import tpu as pltpu
from jax.experimental.pallas import tpu_sc as plsc

NODE = 100
BATCH = 32
DIM = 64
TOPK1 = 20
TOPK2 = 25
N = NODE * BATCH
M = 2 * N
NW = 32
EPT = 2560
NG = EPT // 16
SROWS = 3328
NDUMP = SROWS - N

_HI = lax.Precision.HIGHEST
_BF = jnp.bfloat16


def _bd(a, b):
    return lax.dot_general(a.astype(_BF), b.astype(_BF),
                           (((a.ndim - 1,), (0,)), ((), ())),
                           precision=_HI, preferred_element_type=jnp.float32)


def _bdot(a, b, dims=(((1,), (0,)), ((), ()))):
    return lax.dot_general(a.astype(_BF), b.astype(_BF), dims,
                           preferred_element_type=jnp.float32)


def _ef_body(ep_ref, gcn_ref, out_ref):
    out_ref[...] = lax.dot_general(ep_ref[...], gcn_ref[...],
                                   (((0,), (0,)), ((), ())),
                                   preferred_element_type=jnp.float32)


def _h2_body(gallb_ref, w2_ref, b2_ref, a2i_ref, a2j_ref,
             h2a_ref, ai2_ref, aj2_ref):
    h2 = _bdot(gallb_ref[...], w2_ref[...]) + b2_ref[...]
    ai2_ref[...] = _bdot(h2, a2i_ref[...])
    h2a = h2[0:N, :]
    h2a_ref[...] = h2a
    aj2_ref[...] = _bdot(h2a, a2j_ref[...])


def _mt_body(sp_ref, mt_ref):
    ssum = jnp.sum(sp_ref[...], axis=0, keepdims=True)
    bits = lax.bitcast_convert_type(ssum, jnp.int32)
    e = jnp.bitwise_and(lax.shift_right_logical(bits, 23), 255)
    mt_ref[...] = 4.0 * (e.astype(jnp.float32) - 127.0)


_MESH = plsc.VectorSubcoreMesh(core_axis_name="c", subcore_axis_name="s")
_SC_PARAMS = pltpu.CompilerParams(needs_layout_passes=False)


@functools.partial(
    pl.kernel,
    out_type=[jax.ShapeDtypeStruct((NW, SROWS), jnp.float32),
              jax.ShapeDtypeStruct((NW, EPT), jnp.float32)],
    mesh=_MESH,
    compiler_params=_SC_PARAMS,
    scratch_types=[pltpu.VMEM((EPT,), jnp.int32),
                   pltpu.VMEM((EPT,), jnp.float32),
                   pltpu.VMEM((EPT,), jnp.float32),
                   pltpu.VMEM((M,), jnp.float32),
                   pltpu.VMEM((SROWS,), jnp.float32),
                   pltpu.VMEM((EPT,), jnp.float32)],
)
def _sc_a(dst_hbm, aj_hbm, corr_hbm, ai2_hbm, sp_out, log_out,
          dstv, ajv, corrv, ai2v, sloc, logv):
    c = lax.axis_index("c")
    s = lax.axis_index("s")
    wid = s * 2 + c
    pltpu.sync_copy(dst_hbm.at[wid], dstv)
    pltpu.sync_copy(aj_hbm.at[wid], ajv)
    pltpu.sync_copy(corr_hbm.at[wid], corrv)
    pltpu.sync_copy(ai2_hbm, ai2v)
    zz = jnp.zeros((16,), jnp.float32)

    def zbody(i, _):
        sloc[pl.ds(i * 16, 16)] = zz
        return 0

    lax.fori_loop(0, SROWS // 16, zbody, 0)

    def ebody(g, _):
        sl = pl.ds(g * 16, 16)
        dv = dstv[sl]
        z = ajv[sl] + plsc.load_gather(ai2v, [dv])
        lg = jnp.where(z >= 0, z, 0.2 * z) * corrv[sl]
        logv[sl] = lg
        qt = (lg * 0.25).astype(jnp.int32).astype(jnp.float32)
        pw = jnp.exp(qt * 0.6931471805599453)
        plsc.addupdate_scatter(sloc, [dv], pw)
        return 0

    lax.fori_loop(0, NG, ebody, 0)
    pltpu.sync_copy(logv, log_out.at[wid])
    pltpu.sync_copy(sloc, sp_out.at[wid])


def _fin_body(num_ref, den_ref, out_ref):
    num = num_ref[...]
    den = den_ref[...]
    r = jnp.maximum(num / den, 0.0)
    out_ref[...] = jnp.where(den > 0, r, 0.0)


def kernel(data, org_edge_index, emb_weight, extra_pram, g1_W, g1_b,
           g1_att_i, g1_att_j, bn1_gamma, bn1_beta, g2_W, g2_b,
           g2_att_i, g2_att_j):
    del org_edge_index
    f32 = jnp.float32
    i32 = jnp.int32

    x = data.reshape(-1, data.shape[-1])
    nrm = jnp.linalg.norm(emb_weight, axis=1, keepdims=True)
    cos = _bd(emb_weight, emb_weight.T) / _bd(nrm, nrm.T)
    topk_mat, topk_idx = lax.top_k(cos, TOPK1)
    cos_topk = jnp.tile(topk_mat.reshape(-1), (BATCH,))
    gated_i = jnp.repeat(jnp.arange(NODE), TOPK1)
    gated_j = topk_idx.reshape(-1)
    off = (jnp.arange(BATCH) * NODE)[:, None]
    src1 = (gated_j[None, :] + off).reshape(-1)
    dst1 = (gated_i[None, :] + off).reshape(-1)
    h = _bd(x, g1_W) + g1_b
    g = jnp.concatenate([h, jnp.tile(emb_weight, (BATCH, 1))], axis=-1)
    lg1 = jax.nn.leaky_relu(_bd(g[src1], g1_att_j) + _bd(g[dst1], g1_att_i),
                            0.2) * cos_topk
    m1 = jax.ops.segment_max(lg1, dst1, num_segments=N)
    ex1 = jnp.exp(lg1 - m1[dst1])
    den1 = jax.ops.segment_sum(ex1, dst1, num_segments=N)
    att1 = ex1 / den1[dst1]
    out1 = jax.ops.segment_sum(att1[:, None] * h[src1], dst1, num_segments=N)
    mu = out1.mean(0)
    var = out1.var(0)
    gcn = jax.nn.relu(bn1_gamma * (out1 - mu) / jnp.sqrt(var + 1e-5) + bn1_beta)

    gcn_bf = gcn.astype(_BF)
    ep_bf = extra_pram.astype(_BF)
    ef = pl.pallas_call(
        _ef_body,
        grid=(25,),
        in_specs=[pl.BlockSpec((N, 128), lambda j: (0, j)),
                  pl.BlockSpec((N, DIM), lambda j: (0, 0))],
        out_specs=pl.BlockSpec((128, DIM), lambda j: (j, 0)),
        out_shape=jax.ShapeDtypeStruct((N, DIM), f32),
    )(ep_bf, gcn_bf)

    g_all = jnp.concatenate([gcn, ef], axis=0)
    gall_bf = g_all.astype(_BF)

    nrm2 = jnp.linalg.norm(g_all, axis=-1, keepdims=True)
    D2 = lax.dot_general(gall_bf[:N], gall_bf, (((1,), (1,)), ((), ())),
                         precision=_HI, preferred_element_type=f32)
    P2 = lax.dot_general(nrm2[:N].astype(_BF), nrm2.astype(_BF),
                         (((1,), (1,)), ((), ())),
                         precision=_HI, preferred_element_type=f32)
    tv2, ti2 = lax.top_k(D2 / P2, TOPK2)

    h2a, ai2c, aj2c = pl.pallas_call(
        _h2_body,
        out_shape=[jax.ShapeDtypeStruct((N, DIM), f32),
                   jax.ShapeDtypeStruct((M, 1), f32),
                   jax.ShapeDtypeStruct((N, 1), f32)],
    )(gall_bf, g2_W, g2_b.reshape(1, DIM),
      g2_att_i.reshape(DIM, 1), g2_att_j.reshape(DIM, 1))

    rowid = jnp.arange(N, dtype=i32)
    dump = N + (rowid[:, None] % NDUMP)
    dstm = jnp.where(ti2 < N, ti2, dump)
    ajm = jnp.broadcast_to(aj2c.reshape(N)[:, None], (N, TOPK2))
    paddst = jnp.broadcast_to(
        N + (jnp.arange(EPT - 2500, dtype=i32) % NDUMP)[None, :],
        (NW, EPT - 2500))
    dstE = jnp.concatenate([dstm.reshape(NW, 2500), paddst], axis=1)
    ajE = jnp.concatenate(
        [ajm.reshape(NW, 2500), jnp.zeros((NW, EPT - 2500), f32)], axis=1)
    corrE = jnp.concatenate(
        [tv2.reshape(NW, 2500), jnp.zeros((NW, EPT - 2500), f32)], axis=1)
    src_single = jnp.concatenate(
        [jnp.repeat(jnp.arange(NODE, dtype=i32), TOPK2),
         jnp.zeros((EPT - 2500,), i32)])

    sp, logE = _sc_a(dstE, ajE, corrE, ai2c.reshape(M))

    mt = pl.pallas_call(
        _mt_body,
        out_shape=jax.ShapeDtypeStruct((1, SROWS), f32),
    )(sp).reshape(SROWS)

    ex = jnp.exp(logE - mt[dstE])
    hsrc = h2a.reshape(NW, NODE, DIM)[
        jnp.arange(NW, dtype=i32)[:, None], src_single[None, :]]
    contrib = jnp.concatenate([hsrc * ex[..., None], ex[..., None]], -1)
    accf = jax.ops.segment_sum(contrib.reshape(-1, DIM + 1),
                               dstE.reshape(-1), num_segments=SROWS)

    out = pl.pallas_call(
        _fin_body,
        out_shape=jax.ShapeDtypeStruct((N, DIM), f32),
    )(accf[:N, 0:DIM], accf[:N, DIM:DIM + 1])
    return out
